# Initial kernel scaffold; baseline (speedup 1.0000x reference)
import jax
import jax.numpy as jnp
from jax.experimental import pallas as pl


def kernel(h, edge_index, W1, b1, W2, b2, W3, b3):
    raise NotImplementedError("write your pallas kernel here")


# speedup vs baseline: 30.4593x; 30.4593x over previous
import jax
import jax.numpy as jnp
from jax import lax
from jax.experimental import pallas as pl
from jax.experimental.pallas import tpu as pltpu
from jax.experimental.pallas import tpu_sc as plsc

N_NODES = 100000
N_EDGES = 6400000

NC = 2
NS = 16
NW = NC * NS
CHUNK = 128
G = 8
N_PAD = 100352
DUMMY = N_NODES

E_PAD = 6422528
C_TOTAL = E_PAD // CHUNK
C_TILE = C_TOTAL // NW
ITERS = C_TILE // G

F = 16
BLK = 2048
GRID = N_PAD // BLK

_sc_params = pltpu.CompilerParams(use_tc_tiling_on_sc=False)


def _sc_mesh():
    return plsc.VectorSubcoreMesh(core_axis_name="c", subcore_axis_name="s",
                                  num_cores=NC, num_subcores=NS)


def _edge_pass_body(table, srcc, dstc, zeros, out,
                    src_v, dst_v, rows, acc, gsem, ssem):
    c = lax.axis_index("c")
    s = lax.axis_index("s")
    wid = s * NC + c
    rps = N_PAD // NS
    pltpu.sync_copy(zeros.at[pl.ds(s * rps, rps)], acc.at[pl.ds(s * rps, rps)])
    plsc.subcore_barrier()
    base = wid * C_TILE

    def body(it, carry):
        off = base + it * G
        pltpu.sync_copy(srcc.at[pl.ds(off, G)], src_v)
        pltpu.sync_copy(dstc.at[pl.ds(off, G)], dst_v)
        descs = [pltpu.async_copy(table.at[src_v.at[j]], rows.at[j], gsem)
                 for j in range(G)]
        for d in descs:
            d.wait()
        descs = [pltpu.async_copy(rows.at[j], acc.at[dst_v.at[j]], ssem, add=True)
                 for j in range(G)]
        for d in descs:
            d.wait()
        return carry

    lax.fori_loop(0, ITERS, body, 0)
    plsc.subcore_barrier()
    pltpu.sync_copy(acc.at[pl.ds(s * rps, rps)], out.at[c, pl.ds(s * rps, rps)])


_lazy_cache = {}


def _edge_pass(*args):
    if "edge" not in _lazy_cache:
        _lazy_cache["edge"] = pl.kernel(
            _edge_pass_body,
            out_type=jax.ShapeDtypeStruct((NC, N_PAD, F), jnp.float32),
            mesh=_sc_mesh(),
            compiler_params=_sc_params,
            scratch_types=[
                pltpu.VMEM((G, CHUNK), jnp.int32),
                pltpu.VMEM((G, CHUNK), jnp.int32),
                pltpu.VMEM((G, CHUNK, F), jnp.float32),
                pltpu.VMEM_SHARED((N_PAD, F), jnp.float32),
                pltpu.SemaphoreType.DMA,
                pltpu.SemaphoreType.DMA,
            ],
        )
    return _lazy_cache["edge"](*args)


DW = 8


def _degrees_body(srcc, dstc, ones_hbm, zerosd, dego_out, degi_out,
                  src_v, dst_v, ones_v, dego, degi, sem):
    c = lax.axis_index("c")
    s = lax.axis_index("s")
    wid = s * NC + c
    rps = N_PAD // NS
    pltpu.sync_copy(zerosd.at[pl.ds(s * rps, rps)], dego.at[pl.ds(s * rps, rps)])
    pltpu.sync_copy(zerosd.at[pl.ds(s * rps, rps)], degi.at[pl.ds(s * rps, rps)])
    pltpu.sync_copy(ones_hbm, ones_v)
    plsc.subcore_barrier()
    base = wid * C_TILE

    def body(it, carry):
        off = base + it * G
        pltpu.sync_copy(srcc.at[pl.ds(off, G)], src_v)
        pltpu.sync_copy(dstc.at[pl.ds(off, G)], dst_v)
        descs = [pltpu.async_copy(ones_v, dego.at[src_v.at[j]], sem, add=True)
                 for j in range(G)]
        descs += [pltpu.async_copy(ones_v, degi.at[dst_v.at[j]], sem, add=True)
                  for j in range(G)]
        for d in descs:
            d.wait()
        return carry

    lax.fori_loop(0, ITERS, body, 0)
    plsc.subcore_barrier()
    pltpu.sync_copy(dego.at[pl.ds(s * rps, rps)], dego_out.at[c, pl.ds(s * rps, rps)])
    pltpu.sync_copy(degi.at[pl.ds(s * rps, rps)], degi_out.at[c, pl.ds(s * rps, rps)])


def _degrees(*args):
    if "deg" not in _lazy_cache:
        _lazy_cache["deg"] = pl.kernel(
            _degrees_body,
            out_type=(jax.ShapeDtypeStruct((NC, N_PAD, DW), jnp.float32),
                      jax.ShapeDtypeStruct((NC, N_PAD, DW), jnp.float32)),
            mesh=_sc_mesh(),
            compiler_params=_sc_params,
            scratch_types=[
                pltpu.VMEM((G, CHUNK), jnp.int32),
                pltpu.VMEM((G, CHUNK), jnp.int32),
                pltpu.VMEM((CHUNK, DW), jnp.float32),
                pltpu.VMEM_SHARED((N_PAD, DW), jnp.float32),
                pltpu.VMEM_SHARED((N_PAD, DW), jnp.float32),
                pltpu.SemaphoreType.DMA,
            ],
        )
    return _lazy_cache["deg"](*args)


def _tc1_body(dop_ref, dip_ref, h_ref, w1_ref, y1_ref, ns_ref, nd_ref):
    deg_o = dop_ref[0, :, 0:1] + dop_ref[1, :, 0:1]
    deg_i = dip_ref[0, :, 0:1] + dip_ref[1, :, 0:1]
    ns = jnp.where(deg_o > 0, lax.rsqrt(jnp.maximum(deg_o, 1.0)), 0.0)
    nd = jnp.where(deg_i > 0, lax.rsqrt(jnp.maximum(deg_i, 1.0)), 0.0)
    hs = h_ref[...] * ns
    y1_ref[...] = jnp.dot(hs, w1_ref[...], preferred_element_type=jnp.float32)
    ns_ref[...] = ns
    nd_ref[...] = nd


def _tc2_body(agg_ref, nd_ref, b1_ref, ns_ref, w2_ref, y2_ref):
    x = (agg_ref[0] + agg_ref[1]) * nd_ref[...] + b1_ref[...]
    x = jnp.maximum(x, 0.0)
    y2_ref[...] = jnp.dot(x * ns_ref[...], w2_ref[...],
                          preferred_element_type=jnp.float32)


def _tc3_body(agg_ref, nd_ref, b2_ref, w3_ref, b3_ref, o_ref):
    x = (agg_ref[0] + agg_ref[1]) * nd_ref[...] + b2_ref[...]
    x = jnp.maximum(x, 0.0)
    o_ref[...] = jnp.sum(x * w3_ref[...], axis=1, keepdims=True) + b3_ref[...]


def _part_spec(width):
    return pl.BlockSpec((NC, BLK, width), lambda i: (0, i, 0))


def _row_spec(width):
    return pl.BlockSpec((BLK, width), lambda i: (i, 0))


def _full_spec(shape):
    return pl.BlockSpec(shape, lambda i: tuple(0 for _ in shape))


def kernel(h, edge_index, W1, b1, W2, b2, W3, b3):
    src = edge_index[0]
    dst = edge_index[1]
    pad = E_PAD - N_EDGES
    pad_idx = (N_NODES + jnp.arange(pad, dtype=jnp.int32)
               % (N_PAD - N_NODES)).astype(src.dtype)
    srcc = jnp.concatenate([src, pad_idx]
                           ).reshape(C_TOTAL, CHUNK).astype(jnp.int32)
    dstc = jnp.concatenate([dst, pad_idx]
                           ).reshape(C_TOTAL, CHUNK).astype(jnp.int32)

    h8 = jnp.pad(h, ((0, N_PAD - N_NODES), (0, 2)))
    W1p = jnp.pad(W1, ((0, 2), (0, 0)))
    zeros16 = jnp.zeros((N_PAD, F), jnp.float32)
    zerosd = jnp.zeros((N_PAD, DW), jnp.float32)
    onesd = jnp.zeros((CHUNK, DW), jnp.float32).at[:, 0].set(1.0)

    dego, degi = _degrees(srcc, dstc, onesd, zerosd)

    y1, ns, nd = pl.pallas_call(
        _tc1_body,
        grid=(GRID,),
        in_specs=[_part_spec(DW), _part_spec(DW), _row_spec(8), _full_spec((8, F))],
        out_specs=[_row_spec(F), _row_spec(1), _row_spec(1)],
        out_shape=[jax.ShapeDtypeStruct((N_PAD, F), jnp.float32),
                   jax.ShapeDtypeStruct((N_PAD, 1), jnp.float32),
                   jax.ShapeDtypeStruct((N_PAD, 1), jnp.float32)],
    )(dego, degi, h8, W1p)

    agg1 = _edge_pass(y1, srcc, dstc, zeros16)

    y2 = pl.pallas_call(
        _tc2_body,
        grid=(GRID,),
        in_specs=[_part_spec(F), _row_spec(1), _full_spec((1, F)),
                  _row_spec(1), _full_spec((F, F))],
        out_specs=_row_spec(F),
        out_shape=jax.ShapeDtypeStruct((N_PAD, F), jnp.float32),
    )(agg1, nd, b1.reshape(1, F), ns, W2)

    agg2 = _edge_pass(y2, srcc, dstc, zeros16)

    o = pl.pallas_call(
        _tc3_body,
        grid=(GRID,),
        in_specs=[_part_spec(F), _row_spec(1), _full_spec((1, F)),
                  _full_spec((1, F)), _full_spec((1, 1))],
        out_specs=_row_spec(1),
        out_shape=jax.ShapeDtypeStruct((N_PAD, 1), jnp.float32),
    )(agg2, nd, b2.reshape(1, F), W3.reshape(1, F), b3.reshape(1, 1))

    return o[:N_NODES, 0]


# speedup vs baseline: 34.1307x; 1.1205x over previous
import jax
import jax.numpy as jnp
from jax import lax
from jax.experimental import pallas as pl
from jax.experimental.pallas import tpu as pltpu
from jax.experimental.pallas import tpu_sc as plsc

N_NODES = 100000
N_EDGES = 6400000

NC = 2
NS = 16
NW = NC * NS
CHUNK = 128
G = 4
GD = 8
N_PAD = 100352
DUMMY = N_NODES

E_PAD = 6422528
C_TOTAL = E_PAD // CHUNK
C_TILE = C_TOTAL // NW
ITERS = C_TILE // G
ITERS_D = C_TILE // GD

F = 16
BLK = 2048
GRID = N_PAD // BLK

_sc_params = pltpu.CompilerParams(use_tc_tiling_on_sc=False)


def _sc_mesh():
    return plsc.VectorSubcoreMesh(core_axis_name="c", subcore_axis_name="s",
                                  num_cores=NC, num_subcores=NS)


def _edge_pass_body(table, srcc, dstc, zeros, out,
                    src_v, dst_v, rows, acc, gsem0, gsem1, ssem0, ssem1):
    c = lax.axis_index("c")
    s = lax.axis_index("s")
    wid = s * NC + c
    rps = N_PAD // NS
    pltpu.sync_copy(zeros.at[pl.ds(s * rps, rps)], acc.at[pl.ds(s * rps, rps)])
    plsc.subcore_barrier()
    base = wid * C_TILE
    gsems = (gsem0, gsem1)
    ssems = (ssem0, ssem1)

    def fire_batch(b, slot):
        off = base + b * G
        pltpu.sync_copy(srcc.at[pl.ds(off, G)], src_v.at[slot])
        pltpu.sync_copy(dstc.at[pl.ds(off, G)], dst_v.at[slot])
        for j in range(G):
            pltpu.async_copy(table.at[src_v.at[slot, j]], rows.at[slot, j],
                             gsems[slot])

    def wait_gathers(slot):
        for j in range(G):
            pltpu.make_async_copy(table.at[src_v.at[slot, j]],
                                  rows.at[slot, j], gsems[slot]).wait()

    def fire_scatters(slot):
        for j in range(G):
            pltpu.async_copy(rows.at[slot, j], acc.at[dst_v.at[slot, j]],
                             ssems[slot], add=True)

    def wait_scatters(slot):
        for j in range(G):
            pltpu.make_async_copy(rows.at[slot, j],
                                  acc.at[dst_v.at[slot, j]], ssems[slot]).wait()

    def body(it2, carry):
        b0 = 2 * it2

        @pl.when(it2 > 0)
        def _():
            wait_scatters(0)
        fire_batch(b0, 0)

        @pl.when(it2 > 0)
        def _():
            wait_scatters(1)
        fire_batch(b0 + 1, 1)

        wait_gathers(0)
        fire_scatters(0)
        wait_gathers(1)
        fire_scatters(1)
        return carry

    lax.fori_loop(0, ITERS // 2, body, 0)
    wait_scatters(0)
    wait_scatters(1)
    plsc.subcore_barrier()
    pltpu.sync_copy(acc.at[pl.ds(s * rps, rps)], out.at[c, pl.ds(s * rps, rps)])


_lazy_cache = {}


def _edge_pass(*args):
    if "edge" not in _lazy_cache:
        _lazy_cache["edge"] = pl.kernel(
            _edge_pass_body,
            out_type=jax.ShapeDtypeStruct((NC, N_PAD, F), jnp.float32),
            mesh=_sc_mesh(),
            compiler_params=_sc_params,
            scratch_types=[
                pltpu.VMEM((2, G, CHUNK), jnp.int32),
                pltpu.VMEM((2, G, CHUNK), jnp.int32),
                pltpu.VMEM((2, G, CHUNK, F), jnp.float32),
                pltpu.VMEM_SHARED((N_PAD, F), jnp.float32),
                pltpu.SemaphoreType.DMA,
                pltpu.SemaphoreType.DMA,
                pltpu.SemaphoreType.DMA,
                pltpu.SemaphoreType.DMA,
            ],
        )
    return _lazy_cache["edge"](*args)


DW = 8


def _degrees_body(srcc, dstc, ones_hbm, zerosd, deg_out,
                  src_v, dst_v, ones_v, deg, sem0, sem1):
    c = lax.axis_index("c")
    s = lax.axis_index("s")
    wid = s * NC + c
    rps = N_PAD // NS
    pltpu.sync_copy(zerosd.at[pl.ds(s * rps, rps)], deg.at[pl.ds(s * rps, rps)])
    pltpu.sync_copy(ones_hbm, ones_v)
    plsc.subcore_barrier()
    base = wid * C_TILE
    sems = (sem0, sem1)

    def fire_batch(b, slot):
        off = base + b * GD
        pltpu.sync_copy(srcc.at[pl.ds(off, GD)], src_v.at[slot])
        pltpu.sync_copy(dstc.at[pl.ds(off, GD)], dst_v.at[slot])
        for j in range(GD):
            pltpu.async_copy(ones_v.at[0], deg.at[src_v.at[slot, j]],
                             sems[slot], add=True)
            pltpu.async_copy(ones_v.at[1], deg.at[dst_v.at[slot, j]],
                             sems[slot], add=True)

    def wait_batch(slot):
        for j in range(GD):
            pltpu.make_async_copy(ones_v.at[0], deg.at[src_v.at[slot, j]],
                                  sems[slot]).wait()
            pltpu.make_async_copy(ones_v.at[1], deg.at[dst_v.at[slot, j]],
                                  sems[slot]).wait()

    def body(it2, carry):
        b0 = 2 * it2

        @pl.when(it2 > 0)
        def _():
            wait_batch(0)
        fire_batch(b0, 0)

        @pl.when(it2 > 0)
        def _():
            wait_batch(1)
        fire_batch(b0 + 1, 1)
        return carry

    lax.fori_loop(0, ITERS_D // 2, body, 0)
    wait_batch(0)
    wait_batch(1)
    plsc.subcore_barrier()
    pltpu.sync_copy(deg.at[pl.ds(s * rps, rps)], deg_out.at[c, pl.ds(s * rps, rps)])


def _degrees(*args):
    if "deg" not in _lazy_cache:
        _lazy_cache["deg"] = pl.kernel(
            _degrees_body,
            out_type=jax.ShapeDtypeStruct((NC, N_PAD, DW), jnp.float32),
            mesh=_sc_mesh(),
            compiler_params=_sc_params,
            scratch_types=[
                pltpu.VMEM((2, GD, CHUNK), jnp.int32),
                pltpu.VMEM((2, GD, CHUNK), jnp.int32),
                pltpu.VMEM((2, CHUNK, DW), jnp.float32),
                pltpu.VMEM_SHARED((N_PAD, DW), jnp.float32),
                pltpu.SemaphoreType.DMA,
                pltpu.SemaphoreType.DMA,
            ],
        )
    return _lazy_cache["deg"](*args)


def _tc1_body(deg_ref, h_ref, w1_ref, y1_ref, ns_ref, nd_ref):
    deg_o = deg_ref[0, :, 0:1] + deg_ref[1, :, 0:1]
    deg_i = deg_ref[0, :, 4:5] + deg_ref[1, :, 4:5]
    ns = jnp.where(deg_o > 0, lax.rsqrt(jnp.maximum(deg_o, 1.0)), 0.0)
    nd = jnp.where(deg_i > 0, lax.rsqrt(jnp.maximum(deg_i, 1.0)), 0.0)
    hs = h_ref[...] * ns
    y1_ref[...] = jnp.dot(hs, w1_ref[...], preferred_element_type=jnp.float32)
    ns_ref[...] = ns
    nd_ref[...] = nd


def _tc2_body(agg_ref, nd_ref, b1_ref, ns_ref, w2_ref, y2_ref):
    x = (agg_ref[0] + agg_ref[1]) * nd_ref[...] + b1_ref[...]
    x = jnp.maximum(x, 0.0)
    y2_ref[...] = jnp.dot(x * ns_ref[...], w2_ref[...],
                          preferred_element_type=jnp.float32)


def _tc3_body(agg_ref, nd_ref, b2_ref, w3_ref, b3_ref, o_ref):
    x = (agg_ref[0] + agg_ref[1]) * nd_ref[...] + b2_ref[...]
    x = jnp.maximum(x, 0.0)
    o_ref[...] = jnp.sum(x * w3_ref[...], axis=1, keepdims=True) + b3_ref[...]


def _part_spec(width):
    return pl.BlockSpec((NC, BLK, width), lambda i: (0, i, 0))


def _row_spec(width):
    return pl.BlockSpec((BLK, width), lambda i: (i, 0))


def _full_spec(shape):
    return pl.BlockSpec(shape, lambda i: tuple(0 for _ in shape))


def kernel(h, edge_index, W1, b1, W2, b2, W3, b3):
    src = edge_index[0]
    dst = edge_index[1]
    pad = E_PAD - N_EDGES
    pad_idx = (N_NODES + jnp.arange(pad, dtype=jnp.int32)
               % (N_PAD - N_NODES)).astype(src.dtype)
    srcc = jnp.concatenate([src, pad_idx]
                           ).reshape(C_TOTAL, CHUNK).astype(jnp.int32)
    dstc = jnp.concatenate([dst, pad_idx]
                           ).reshape(C_TOTAL, CHUNK).astype(jnp.int32)

    h8 = jnp.pad(h, ((0, N_PAD - N_NODES), (0, 2)))
    W1p = jnp.pad(W1, ((0, 2), (0, 0)))
    zeros16 = jnp.zeros((N_PAD, F), jnp.float32)
    zerosd = jnp.zeros((N_PAD, DW), jnp.float32)
    onesd = (jnp.zeros((2, CHUNK, DW), jnp.float32)
             .at[0, :, 0].set(1.0).at[1, :, 4].set(1.0))

    deg = _degrees(srcc, dstc, onesd, zerosd)

    y1, ns, nd = pl.pallas_call(
        _tc1_body,
        grid=(GRID,),
        in_specs=[_part_spec(DW), _row_spec(8), _full_spec((8, F))],
        out_specs=[_row_spec(F), _row_spec(1), _row_spec(1)],
        out_shape=[jax.ShapeDtypeStruct((N_PAD, F), jnp.float32),
                   jax.ShapeDtypeStruct((N_PAD, 1), jnp.float32),
                   jax.ShapeDtypeStruct((N_PAD, 1), jnp.float32)],
    )(deg, h8, W1p)

    agg1 = _edge_pass(y1, srcc, dstc, zeros16)

    y2 = pl.pallas_call(
        _tc2_body,
        grid=(GRID,),
        in_specs=[_part_spec(F), _row_spec(1), _full_spec((1, F)),
                  _row_spec(1), _full_spec((F, F))],
        out_specs=_row_spec(F),
        out_shape=jax.ShapeDtypeStruct((N_PAD, F), jnp.float32),
    )(agg1, nd, b1.reshape(1, F), ns, W2)

    agg2 = _edge_pass(y2, srcc, dstc, zeros16)

    o = pl.pallas_call(
        _tc3_body,
        grid=(GRID,),
        in_specs=[_part_spec(F), _row_spec(1), _full_spec((1, F)),
                  _full_spec((1, F)), _full_spec((1, 1))],
        out_specs=_row_spec(1),
        out_shape=jax.ShapeDtypeStruct((N_PAD, 1), jnp.float32),
    )(agg2, nd, b2.reshape(1, F), W3.reshape(1, F), b3.reshape(1, 1))

    return o[:N_NODES, 0]


# speedup vs baseline: 36.8210x; 1.0788x over previous
import jax
import jax.numpy as jnp
from jax import lax
from jax.experimental import pallas as pl
from jax.experimental.pallas import tpu as pltpu
from jax.experimental.pallas import tpu_sc as plsc

N_NODES = 100000
N_EDGES = 6400000

NC = 2
NS = 16
NW = NC * NS
CHUNK = 128
G = 4
GD = 8
N_PAD = 100352
DUMMY = N_NODES

E_PAD = 6422528
C_TOTAL = E_PAD // CHUNK
C_TILE = C_TOTAL // NW
ITERS = C_TILE // G
ITERS_D = C_TILE // GD

F = 16
BLK = 2048
GRID = N_PAD // BLK

_sc_params = pltpu.CompilerParams(use_tc_tiling_on_sc=False)


def _sc_mesh():
    return plsc.VectorSubcoreMesh(core_axis_name="c", subcore_axis_name="s",
                                  num_cores=NC, num_subcores=NS)


def _make_edge_pass_body(width, g, iters):
    def _edge_pass_body(table, srcc, dstc, zeros, out,
                        src_v, dst_v, rows, acc, gsem0, gsem1, ssem0, ssem1):
        c = lax.axis_index("c")
        s = lax.axis_index("s")
        wid = s * NC + c
        rps = N_PAD // NS
        pltpu.sync_copy(zeros.at[pl.ds(s * rps, rps)],
                        acc.at[pl.ds(s * rps, rps)])
        plsc.subcore_barrier()
        base = wid * C_TILE
        gsems = (gsem0, gsem1)
        ssems = (ssem0, ssem1)

        def fire_batch(b, slot):
            off = base + b * g
            pltpu.sync_copy(srcc.at[pl.ds(off, g)], src_v.at[slot])
            pltpu.sync_copy(dstc.at[pl.ds(off, g)], dst_v.at[slot])
            for j in range(g):
                pltpu.async_copy(table.at[src_v.at[slot, j]], rows.at[slot, j],
                                 gsems[slot])

        def wait_gathers(slot):
            for j in range(g):
                pltpu.make_async_copy(table.at[src_v.at[slot, j]],
                                      rows.at[slot, j], gsems[slot]).wait()

        def fire_scatters(slot):
            for j in range(g):
                pltpu.async_copy(rows.at[slot, j], acc.at[dst_v.at[slot, j]],
                                 ssems[slot], add=True)

        def wait_scatters(slot):
            for j in range(g):
                pltpu.make_async_copy(rows.at[slot, j],
                                      acc.at[dst_v.at[slot, j]],
                                      ssems[slot]).wait()

        def body(it2, carry):
            b0 = 2 * it2

            @pl.when(it2 > 0)
            def _():
                wait_scatters(0)
            fire_batch(b0, 0)

            @pl.when(it2 > 0)
            def _():
                wait_scatters(1)
            fire_batch(b0 + 1, 1)

            wait_gathers(0)
            fire_scatters(0)
            wait_gathers(1)
            fire_scatters(1)
            return carry

        lax.fori_loop(0, iters // 2, body, 0)
        wait_scatters(0)
        wait_scatters(1)
        plsc.subcore_barrier()
        pltpu.sync_copy(acc.at[pl.ds(s * rps, rps)],
                        out.at[c, pl.ds(s * rps, rps)])

    return _edge_pass_body


_lazy_cache = {}


def _edge_pass(width, g, *args):
    key = ("edge", width)
    if key not in _lazy_cache:
        _lazy_cache[key] = pl.kernel(
            _make_edge_pass_body(width, g, C_TILE // g),
            out_type=jax.ShapeDtypeStruct((NC, N_PAD, width), jnp.float32),
            mesh=_sc_mesh(),
            compiler_params=_sc_params,
            scratch_types=[
                pltpu.VMEM((2, g, CHUNK), jnp.int32),
                pltpu.VMEM((2, g, CHUNK), jnp.int32),
                pltpu.VMEM((2, g, CHUNK, width), jnp.float32),
                pltpu.VMEM_SHARED((N_PAD, width), jnp.float32),
                pltpu.SemaphoreType.DMA,
                pltpu.SemaphoreType.DMA,
                pltpu.SemaphoreType.DMA,
                pltpu.SemaphoreType.DMA,
            ],
        )
    return _lazy_cache[key](*args)


DW = 8


def _degrees_body(srcc, dstc, ones_hbm, zerosd, deg_out,
                  src_v, dst_v, ones_v, deg, sem0, sem1):
    c = lax.axis_index("c")
    s = lax.axis_index("s")
    wid = s * NC + c
    rps = N_PAD // NS
    pltpu.sync_copy(zerosd.at[pl.ds(s * rps, rps)], deg.at[pl.ds(s * rps, rps)])
    pltpu.sync_copy(ones_hbm, ones_v)
    plsc.subcore_barrier()
    base = wid * C_TILE
    sems = (sem0, sem1)

    def fire_batch(b, slot):
        off = base + b * GD
        pltpu.sync_copy(srcc.at[pl.ds(off, GD)], src_v.at[slot])
        pltpu.sync_copy(dstc.at[pl.ds(off, GD)], dst_v.at[slot])
        for j in range(GD):
            pltpu.async_copy(ones_v.at[0], deg.at[src_v.at[slot, j]],
                             sems[slot], add=True)
            pltpu.async_copy(ones_v.at[1], deg.at[dst_v.at[slot, j]],
                             sems[slot], add=True)

    def wait_batch(slot):
        for j in range(GD):
            pltpu.make_async_copy(ones_v.at[0], deg.at[src_v.at[slot, j]],
                                  sems[slot]).wait()
            pltpu.make_async_copy(ones_v.at[1], deg.at[dst_v.at[slot, j]],
                                  sems[slot]).wait()

    def body(it2, carry):
        b0 = 2 * it2

        @pl.when(it2 > 0)
        def _():
            wait_batch(0)
        fire_batch(b0, 0)

        @pl.when(it2 > 0)
        def _():
            wait_batch(1)
        fire_batch(b0 + 1, 1)
        return carry

    lax.fori_loop(0, ITERS_D // 2, body, 0)
    wait_batch(0)
    wait_batch(1)
    plsc.subcore_barrier()
    pltpu.sync_copy(deg.at[pl.ds(s * rps, rps)], deg_out.at[c, pl.ds(s * rps, rps)])


def _degrees(*args):
    if "deg" not in _lazy_cache:
        _lazy_cache["deg"] = pl.kernel(
            _degrees_body,
            out_type=jax.ShapeDtypeStruct((NC, N_PAD, DW), jnp.float32),
            mesh=_sc_mesh(),
            compiler_params=_sc_params,
            scratch_types=[
                pltpu.VMEM((2, GD, CHUNK), jnp.int32),
                pltpu.VMEM((2, GD, CHUNK), jnp.int32),
                pltpu.VMEM((2, CHUNK, DW), jnp.float32),
                pltpu.VMEM_SHARED((N_PAD, DW), jnp.float32),
                pltpu.SemaphoreType.DMA,
                pltpu.SemaphoreType.DMA,
            ],
        )
    return _lazy_cache["deg"](*args)


def _tc1_body(deg_ref, h_ref, y1_ref, ns_ref, nd_ref):
    deg_o = deg_ref[0, :, 0:1] + deg_ref[1, :, 0:1]
    deg_i = deg_ref[0, :, 4:5] + deg_ref[1, :, 4:5]
    ns = jnp.where(deg_o > 0, lax.rsqrt(jnp.maximum(deg_o, 1.0)), 0.0)
    nd = jnp.where(deg_i > 0, lax.rsqrt(jnp.maximum(deg_i, 1.0)), 0.0)
    y1_ref[...] = h_ref[...] * ns
    ns_ref[...] = ns
    nd_ref[...] = nd


def _tc2_body(agg_ref, nd_ref, b1_ref, ns_ref, w1_ref, w2_ref, y2_ref):
    x = jnp.dot((agg_ref[0] + agg_ref[1]) * nd_ref[...], w1_ref[...],
                preferred_element_type=jnp.float32,
                precision=lax.Precision.HIGHEST) + b1_ref[...]
    x = jnp.maximum(x, 0.0)
    y2_ref[...] = jnp.dot(x * ns_ref[...], w2_ref[...],
                          preferred_element_type=jnp.float32,
                          precision=lax.Precision.HIGHEST)


def _tc3_body(agg_ref, nd_ref, b2_ref, w3_ref, b3_ref, o_ref):
    x = (agg_ref[0] + agg_ref[1]) * nd_ref[...] + b2_ref[...]
    x = jnp.maximum(x, 0.0)
    o_ref[...] = jnp.sum(x * w3_ref[...], axis=1, keepdims=True) + b3_ref[...]


def _part_spec(width):
    return pl.BlockSpec((NC, BLK, width), lambda i: (0, i, 0))


def _row_spec(width):
    return pl.BlockSpec((BLK, width), lambda i: (i, 0))


def _full_spec(shape):
    return pl.BlockSpec(shape, lambda i: tuple(0 for _ in shape))


def kernel(h, edge_index, W1, b1, W2, b2, W3, b3):
    src = edge_index[0]
    dst = edge_index[1]
    pad = E_PAD - N_EDGES
    pad_idx = (N_NODES + jnp.arange(pad, dtype=jnp.int32)
               % (N_PAD - N_NODES)).astype(src.dtype)
    srcc = jnp.concatenate([src, pad_idx]
                           ).reshape(C_TOTAL, CHUNK).astype(jnp.int32)
    dstc = jnp.concatenate([dst, pad_idx]
                           ).reshape(C_TOTAL, CHUNK).astype(jnp.int32)

    h8 = jnp.pad(h, ((0, N_PAD - N_NODES), (0, 2)))
    W1p = jnp.pad(W1, ((0, 2), (0, 0)))
    zeros16 = jnp.zeros((N_PAD, F), jnp.float32)
    zeros8 = jnp.zeros((N_PAD, 8), jnp.float32)
    zerosd = jnp.zeros((N_PAD, DW), jnp.float32)
    onesd = (jnp.zeros((2, CHUNK, DW), jnp.float32)
             .at[0, :, 0].set(1.0).at[1, :, 4].set(1.0))

    deg = _degrees(srcc, dstc, onesd, zerosd)

    y1, ns, nd = pl.pallas_call(
        _tc1_body,
        grid=(GRID,),
        in_specs=[_part_spec(DW), _row_spec(8)],
        out_specs=[_row_spec(8), _row_spec(1), _row_spec(1)],
        out_shape=[jax.ShapeDtypeStruct((N_PAD, 8), jnp.float32),
                   jax.ShapeDtypeStruct((N_PAD, 1), jnp.float32),
                   jax.ShapeDtypeStruct((N_PAD, 1), jnp.float32)],
    )(deg, h8)

    agg1 = _edge_pass(8, GD, y1, srcc, dstc, zeros8)

    y2 = pl.pallas_call(
        _tc2_body,
        grid=(GRID,),
        in_specs=[_part_spec(8), _row_spec(1), _full_spec((1, F)),
                  _row_spec(1), _full_spec((8, F)), _full_spec((F, F))],
        out_specs=_row_spec(F),
        out_shape=jax.ShapeDtypeStruct((N_PAD, F), jnp.float32),
    )(agg1, nd, b1.reshape(1, F), ns, W1p, W2)

    agg2 = _edge_pass(F, G, y2, srcc, dstc, zeros16)

    o = pl.pallas_call(
        _tc3_body,
        grid=(GRID,),
        in_specs=[_part_spec(F), _row_spec(1), _full_spec((1, F)),
                  _full_spec((1, F)), _full_spec((1, 1))],
        out_specs=_row_spec(1),
        out_shape=jax.ShapeDtypeStruct((N_PAD, 1), jnp.float32),
    )(agg2, nd, b2.reshape(1, F), W3.reshape(1, F), b3.reshape(1, 1))

    return o[:N_NODES, 0]


# speedup vs baseline: 36.8347x; 1.0004x over previous
import jax
import jax.numpy as jnp
from jax import lax
from jax.experimental import pallas as pl
from jax.experimental.pallas import tpu as pltpu
from jax.experimental.pallas import tpu_sc as plsc

N_NODES = 100000
N_EDGES = 6400000

NC = 2
NS = 16
NW = NC * NS
CHUNK = 128
G = 4
GD = 8
N_PAD = 100352
DUMMY = N_NODES

E_PAD = 6422528
C_TOTAL = E_PAD // CHUNK
C_TILE = C_TOTAL // NW
ITERS = C_TILE // G
ITERS_D = C_TILE // GD

F = 16
BLK = 2048
GRID = N_PAD // BLK

_sc_params = pltpu.CompilerParams(use_tc_tiling_on_sc=False)


def _sc_mesh():
    return plsc.VectorSubcoreMesh(core_axis_name="c", subcore_axis_name="s",
                                  num_cores=NC, num_subcores=NS)


def _make_edge_pass_body(width, g, iters):
    gc = g * CHUNK

    def _edge_pass_body(table, srcf, dstc, zeros, out,
                        src_v, dst_v, rows, acc, gsem0, gsem1, ssem0, ssem1):
        c = lax.axis_index("c")
        s = lax.axis_index("s")
        wid = s * NC + c
        rps = N_PAD // NS
        pltpu.sync_copy(zeros.at[pl.ds(s * rps, rps)],
                        acc.at[pl.ds(s * rps, rps)])
        plsc.subcore_barrier()
        base = wid * C_TILE
        gsems = (gsem0, gsem1)
        ssems = (ssem0, ssem1)

        def fire_batch(b, slot):
            pltpu.sync_copy(srcf.at[pl.ds((base + b * g) * CHUNK, gc)],
                            src_v.at[slot])
            pltpu.sync_copy(dstc.at[pl.ds(base + b * g, g)], dst_v.at[slot])
            pltpu.async_copy(table.at[src_v.at[slot]], rows.at[slot],
                             gsems[slot])

        def wait_gathers(slot):
            pltpu.make_async_copy(table.at[src_v.at[slot]],
                                  rows.at[slot], gsems[slot]).wait()

        def fire_scatters(slot):
            for j in range(g):
                pltpu.async_copy(rows.at[slot, pl.ds(j * CHUNK, CHUNK)],
                                 acc.at[dst_v.at[slot, j]],
                                 ssems[slot], add=True)

        def wait_scatters(slot):
            for j in range(g):
                pltpu.make_async_copy(rows.at[slot, pl.ds(j * CHUNK, CHUNK)],
                                      acc.at[dst_v.at[slot, j]],
                                      ssems[slot]).wait()

        def body(it2, carry):
            b0 = 2 * it2

            @pl.when(it2 > 0)
            def _():
                wait_scatters(0)
            fire_batch(b0, 0)

            @pl.when(it2 > 0)
            def _():
                wait_scatters(1)
            fire_batch(b0 + 1, 1)

            wait_gathers(0)
            fire_scatters(0)
            wait_gathers(1)
            fire_scatters(1)
            return carry

        lax.fori_loop(0, iters // 2, body, 0)
        wait_scatters(0)
        wait_scatters(1)
        plsc.subcore_barrier()
        pltpu.sync_copy(acc.at[pl.ds(s * rps, rps)],
                        out.at[c, pl.ds(s * rps, rps)])

    return _edge_pass_body


_lazy_cache = {}


def _edge_pass(width, g, *args):
    key = ("edge", width)
    if key not in _lazy_cache:
        _lazy_cache[key] = pl.kernel(
            _make_edge_pass_body(width, g, C_TILE // g),
            out_type=jax.ShapeDtypeStruct((NC, N_PAD, width), jnp.float32),
            mesh=_sc_mesh(),
            compiler_params=_sc_params,
            scratch_types=[
                pltpu.VMEM((2, g * CHUNK), jnp.int32),
                pltpu.VMEM((2, g, CHUNK), jnp.int32),
                pltpu.VMEM((2, g * CHUNK, width), jnp.float32),
                pltpu.VMEM_SHARED((N_PAD, width), jnp.float32),
                pltpu.SemaphoreType.DMA,
                pltpu.SemaphoreType.DMA,
                pltpu.SemaphoreType.DMA,
                pltpu.SemaphoreType.DMA,
            ],
        )
    return _lazy_cache[key](*args)


DW = 8


def _degrees_body(srcc, dstc, ones_hbm, zerosd, deg_out,
                  src_v, dst_v, ones_v, deg, sem0, sem1):
    c = lax.axis_index("c")
    s = lax.axis_index("s")
    wid = s * NC + c
    rps = N_PAD // NS
    pltpu.sync_copy(zerosd.at[pl.ds(s * rps, rps)], deg.at[pl.ds(s * rps, rps)])
    pltpu.sync_copy(ones_hbm, ones_v)
    plsc.subcore_barrier()
    base = wid * C_TILE
    sems = (sem0, sem1)

    def fire_batch(b, slot):
        off = base + b * GD
        pltpu.sync_copy(srcc.at[pl.ds(off, GD)], src_v.at[slot])
        pltpu.sync_copy(dstc.at[pl.ds(off, GD)], dst_v.at[slot])
        for j in range(GD):
            pltpu.async_copy(ones_v.at[0], deg.at[src_v.at[slot, j]],
                             sems[slot], add=True)
            pltpu.async_copy(ones_v.at[1], deg.at[dst_v.at[slot, j]],
                             sems[slot], add=True)

    def wait_batch(slot):
        for j in range(GD):
            pltpu.make_async_copy(ones_v.at[0], deg.at[src_v.at[slot, j]],
                                  sems[slot]).wait()
            pltpu.make_async_copy(ones_v.at[1], deg.at[dst_v.at[slot, j]],
                                  sems[slot]).wait()

    def body(it2, carry):
        b0 = 2 * it2

        @pl.when(it2 > 0)
        def _():
            wait_batch(0)
        fire_batch(b0, 0)

        @pl.when(it2 > 0)
        def _():
            wait_batch(1)
        fire_batch(b0 + 1, 1)
        return carry

    lax.fori_loop(0, ITERS_D // 2, body, 0)
    wait_batch(0)
    wait_batch(1)
    plsc.subcore_barrier()
    pltpu.sync_copy(deg.at[pl.ds(s * rps, rps)], deg_out.at[c, pl.ds(s * rps, rps)])


def _degrees(*args):
    if "deg" not in _lazy_cache:
        _lazy_cache["deg"] = pl.kernel(
            _degrees_body,
            out_type=jax.ShapeDtypeStruct((NC, N_PAD, DW), jnp.float32),
            mesh=_sc_mesh(),
            compiler_params=_sc_params,
            scratch_types=[
                pltpu.VMEM((2, GD, CHUNK), jnp.int32),
                pltpu.VMEM((2, GD, CHUNK), jnp.int32),
                pltpu.VMEM((2, CHUNK, DW), jnp.float32),
                pltpu.VMEM_SHARED((N_PAD, DW), jnp.float32),
                pltpu.SemaphoreType.DMA,
                pltpu.SemaphoreType.DMA,
            ],
        )
    return _lazy_cache["deg"](*args)


def _tc1_body(deg_ref, h_ref, y1_ref, ns_ref, nd_ref):
    deg_o = deg_ref[0, :, 0:1] + deg_ref[1, :, 0:1]
    deg_i = deg_ref[0, :, 4:5] + deg_ref[1, :, 4:5]
    ns = jnp.where(deg_o > 0, lax.rsqrt(jnp.maximum(deg_o, 1.0)), 0.0)
    nd = jnp.where(deg_i > 0, lax.rsqrt(jnp.maximum(deg_i, 1.0)), 0.0)
    y1_ref[...] = h_ref[...] * ns
    ns_ref[...] = ns
    nd_ref[...] = nd


def _tc2_body(agg_ref, nd_ref, b1_ref, ns_ref, w1_ref, w2_ref, y2_ref):
    x = jnp.dot((agg_ref[0] + agg_ref[1]) * nd_ref[...], w1_ref[...],
                preferred_element_type=jnp.float32,
                precision=lax.Precision.HIGHEST) + b1_ref[...]
    x = jnp.maximum(x, 0.0)
    y2_ref[...] = jnp.dot(x * ns_ref[...], w2_ref[...],
                          preferred_element_type=jnp.float32,
                          precision=lax.Precision.HIGHEST)


def _tc3_body(agg_ref, nd_ref, b2_ref, w3_ref, b3_ref, o_ref):
    x = (agg_ref[0] + agg_ref[1]) * nd_ref[...] + b2_ref[...]
    x = jnp.maximum(x, 0.0)
    o_ref[...] = jnp.sum(x * w3_ref[...], axis=1, keepdims=True) + b3_ref[...]


def _part_spec(width):
    return pl.BlockSpec((NC, BLK, width), lambda i: (0, i, 0))


def _row_spec(width):
    return pl.BlockSpec((BLK, width), lambda i: (i, 0))


def _full_spec(shape):
    return pl.BlockSpec(shape, lambda i: tuple(0 for _ in shape))


def kernel(h, edge_index, W1, b1, W2, b2, W3, b3):
    src = edge_index[0]
    dst = edge_index[1]
    pad = E_PAD - N_EDGES
    pad_idx = (N_NODES + jnp.arange(pad, dtype=jnp.int32)
               % (N_PAD - N_NODES)).astype(src.dtype)
    srcf = jnp.concatenate([src, pad_idx]).astype(jnp.int32)
    srcc = srcf.reshape(C_TOTAL, CHUNK)
    dstc = jnp.concatenate([dst, pad_idx]
                           ).reshape(C_TOTAL, CHUNK).astype(jnp.int32)

    h8 = jnp.pad(h, ((0, N_PAD - N_NODES), (0, 2)))
    W1p = jnp.pad(W1, ((0, 2), (0, 0)))
    zeros16 = jnp.zeros((N_PAD, F), jnp.float32)
    zeros8 = jnp.zeros((N_PAD, 8), jnp.float32)
    zerosd = jnp.zeros((N_PAD, DW), jnp.float32)
    onesd = (jnp.zeros((2, CHUNK, DW), jnp.float32)
             .at[0, :, 0].set(1.0).at[1, :, 4].set(1.0))

    deg = _degrees(srcc, dstc, onesd, zerosd)

    y1, ns, nd = pl.pallas_call(
        _tc1_body,
        grid=(GRID,),
        in_specs=[_part_spec(DW), _row_spec(8)],
        out_specs=[_row_spec(8), _row_spec(1), _row_spec(1)],
        out_shape=[jax.ShapeDtypeStruct((N_PAD, 8), jnp.float32),
                   jax.ShapeDtypeStruct((N_PAD, 1), jnp.float32),
                   jax.ShapeDtypeStruct((N_PAD, 1), jnp.float32)],
    )(deg, h8)

    agg1 = _edge_pass(8, GD, y1, srcf, dstc, zeros8)

    y2 = pl.pallas_call(
        _tc2_body,
        grid=(GRID,),
        in_specs=[_part_spec(8), _row_spec(1), _full_spec((1, F)),
                  _row_spec(1), _full_spec((8, F)), _full_spec((F, F))],
        out_specs=_row_spec(F),
        out_shape=jax.ShapeDtypeStruct((N_PAD, F), jnp.float32),
    )(agg1, nd, b1.reshape(1, F), ns, W1p, W2)

    agg2 = _edge_pass(F, G, y2, srcf, dstc, zeros16)

    o = pl.pallas_call(
        _tc3_body,
        grid=(GRID,),
        in_specs=[_part_spec(F), _row_spec(1), _full_spec((1, F)),
                  _full_spec((1, F)), _full_spec((1, 1))],
        out_specs=_row_spec(1),
        out_shape=jax.ShapeDtypeStruct((N_PAD, 1), jnp.float32),
    )(agg2, nd, b2.reshape(1, F), W3.reshape(1, F), b3.reshape(1, 1))

    return o[:N_NODES, 0]


# speedup vs baseline: 36.9951x; 1.0044x over previous
import jax
import jax.numpy as jnp
from jax import lax
from jax.experimental import pallas as pl
from jax.experimental.pallas import tpu as pltpu
from jax.experimental.pallas import tpu_sc as plsc

N_NODES = 100000
N_EDGES = 6400000

NC = 2
NS = 16
NW = NC * NS
CHUNK = 128
G = 4
GD = 8
N_PAD = 100352
DUMMY = N_NODES

E_PAD = 6422528
C_TOTAL = E_PAD // CHUNK
C_TILE = C_TOTAL // NW
ITERS = C_TILE // G
ITERS_D = C_TILE // GD

F = 16
BLK = 2048
GRID = N_PAD // BLK

_sc_params = pltpu.CompilerParams(use_tc_tiling_on_sc=False)


def _sc_mesh():
    return plsc.VectorSubcoreMesh(core_axis_name="c", subcore_axis_name="s",
                                  num_cores=NC, num_subcores=NS)


def _make_edge_pass_body(width, g, iters):
    gc = g * CHUNK

    def _edge_pass_body(table, srcf, dstf, zeros, out,
                        src_v, dst_v, rows, acc, gsem0, gsem1, ssem0, ssem1):
        c = lax.axis_index("c")
        s = lax.axis_index("s")
        wid = s * NC + c
        rps = N_PAD // NS
        pltpu.sync_copy(zeros.at[pl.ds(s * rps, rps)],
                        acc.at[pl.ds(s * rps, rps)])
        plsc.subcore_barrier()
        base = wid * C_TILE
        gsems = (gsem0, gsem1)
        ssems = (ssem0, ssem1)

        def fire_batch(b, slot):
            pltpu.sync_copy(srcf.at[pl.ds((base + b * g) * CHUNK, gc)],
                            src_v.at[slot])
            pltpu.sync_copy(dstf.at[pl.ds((base + b * g) * CHUNK, gc)],
                            dst_v.at[slot])
            pltpu.async_copy(table.at[src_v.at[slot]], rows.at[slot],
                             gsems[slot])

        def wait_gathers(slot):
            pltpu.make_async_copy(table.at[src_v.at[slot]],
                                  rows.at[slot], gsems[slot]).wait()

        def fire_scatters(slot):
            pltpu.async_copy(rows.at[slot], acc.at[dst_v.at[slot]],
                             ssems[slot], add=True)

        def wait_scatters(slot):
            pltpu.make_async_copy(rows.at[slot], acc.at[dst_v.at[slot]],
                                  ssems[slot]).wait()

        def body(it2, carry):
            b0 = 2 * it2

            @pl.when(it2 > 0)
            def _():
                wait_scatters(0)
            fire_batch(b0, 0)

            @pl.when(it2 > 0)
            def _():
                wait_scatters(1)
            fire_batch(b0 + 1, 1)

            wait_gathers(0)
            fire_scatters(0)
            wait_gathers(1)
            fire_scatters(1)
            return carry

        lax.fori_loop(0, iters // 2, body, 0)
        wait_scatters(0)
        wait_scatters(1)
        plsc.subcore_barrier()
        pltpu.sync_copy(acc.at[pl.ds(s * rps, rps)],
                        out.at[c, pl.ds(s * rps, rps)])

    return _edge_pass_body


_lazy_cache = {}


def _edge_pass(width, g, *args):
    key = ("edge", width)
    if key not in _lazy_cache:
        _lazy_cache[key] = pl.kernel(
            _make_edge_pass_body(width, g, C_TILE // g),
            out_type=jax.ShapeDtypeStruct((NC, N_PAD, width), jnp.float32),
            mesh=_sc_mesh(),
            compiler_params=_sc_params,
            scratch_types=[
                pltpu.VMEM((2, g * CHUNK), jnp.int32),
                pltpu.VMEM((2, g * CHUNK), jnp.int32),
                pltpu.VMEM((2, g * CHUNK, width), jnp.float32),
                pltpu.VMEM_SHARED((N_PAD, width), jnp.float32),
                pltpu.SemaphoreType.DMA,
                pltpu.SemaphoreType.DMA,
                pltpu.SemaphoreType.DMA,
                pltpu.SemaphoreType.DMA,
            ],
        )
    return _lazy_cache[key](*args)


DW = 8


def _degrees_body(srcc, dstc, ones_hbm, zerosd, deg_out,
                  src_v, dst_v, ones_v, deg, sem0, sem1):
    c = lax.axis_index("c")
    s = lax.axis_index("s")
    wid = s * NC + c
    rps = N_PAD // NS
    pltpu.sync_copy(zerosd.at[pl.ds(s * rps, rps)], deg.at[pl.ds(s * rps, rps)])
    pltpu.sync_copy(ones_hbm, ones_v)
    plsc.subcore_barrier()
    base = wid * C_TILE
    sems = (sem0, sem1)

    def fire_batch(b, slot):
        off = base + b * GD
        pltpu.sync_copy(srcc.at[pl.ds(off, GD)], src_v.at[slot])
        pltpu.sync_copy(dstc.at[pl.ds(off, GD)], dst_v.at[slot])
        for j in range(GD):
            pltpu.async_copy(ones_v.at[0], deg.at[src_v.at[slot, j]],
                             sems[slot], add=True)
            pltpu.async_copy(ones_v.at[1], deg.at[dst_v.at[slot, j]],
                             sems[slot], add=True)

    def wait_batch(slot):
        for j in range(GD):
            pltpu.make_async_copy(ones_v.at[0], deg.at[src_v.at[slot, j]],
                                  sems[slot]).wait()
            pltpu.make_async_copy(ones_v.at[1], deg.at[dst_v.at[slot, j]],
                                  sems[slot]).wait()

    def body(it2, carry):
        b0 = 2 * it2

        @pl.when(it2 > 0)
        def _():
            wait_batch(0)
        fire_batch(b0, 0)

        @pl.when(it2 > 0)
        def _():
            wait_batch(1)
        fire_batch(b0 + 1, 1)
        return carry

    lax.fori_loop(0, ITERS_D // 2, body, 0)
    wait_batch(0)
    wait_batch(1)
    plsc.subcore_barrier()
    pltpu.sync_copy(deg.at[pl.ds(s * rps, rps)], deg_out.at[c, pl.ds(s * rps, rps)])


def _degrees(*args):
    if "deg" not in _lazy_cache:
        _lazy_cache["deg"] = pl.kernel(
            _degrees_body,
            out_type=jax.ShapeDtypeStruct((NC, N_PAD, DW), jnp.float32),
            mesh=_sc_mesh(),
            compiler_params=_sc_params,
            scratch_types=[
                pltpu.VMEM((2, GD, CHUNK), jnp.int32),
                pltpu.VMEM((2, GD, CHUNK), jnp.int32),
                pltpu.VMEM((2, CHUNK, DW), jnp.float32),
                pltpu.VMEM_SHARED((N_PAD, DW), jnp.float32),
                pltpu.SemaphoreType.DMA,
                pltpu.SemaphoreType.DMA,
            ],
        )
    return _lazy_cache["deg"](*args)


def _tc1_body(deg_ref, h_ref, y1_ref, ns_ref, nd_ref):
    deg_o = deg_ref[0, :, 0:1] + deg_ref[1, :, 0:1]
    deg_i = deg_ref[0, :, 4:5] + deg_ref[1, :, 4:5]
    ns = jnp.where(deg_o > 0, lax.rsqrt(jnp.maximum(deg_o, 1.0)), 0.0)
    nd = jnp.where(deg_i > 0, lax.rsqrt(jnp.maximum(deg_i, 1.0)), 0.0)
    y1_ref[...] = h_ref[...] * ns
    ns_ref[...] = ns
    nd_ref[...] = nd


def _tc2_body(agg_ref, nd_ref, b1_ref, ns_ref, w1_ref, w2_ref, y2_ref):
    x = jnp.dot((agg_ref[0] + agg_ref[1]) * nd_ref[...], w1_ref[...],
                preferred_element_type=jnp.float32,
                precision=lax.Precision.HIGHEST) + b1_ref[...]
    x = jnp.maximum(x, 0.0)
    y2_ref[...] = jnp.dot(x * ns_ref[...], w2_ref[...],
                          preferred_element_type=jnp.float32,
                          precision=lax.Precision.HIGHEST)


def _tc3_body(agg_ref, nd_ref, b2_ref, w3_ref, b3_ref, o_ref):
    x = (agg_ref[0] + agg_ref[1]) * nd_ref[...] + b2_ref[...]
    x = jnp.maximum(x, 0.0)
    o_ref[...] = jnp.sum(x * w3_ref[...], axis=1, keepdims=True) + b3_ref[...]


def _part_spec(width):
    return pl.BlockSpec((NC, BLK, width), lambda i: (0, i, 0))


def _row_spec(width):
    return pl.BlockSpec((BLK, width), lambda i: (i, 0))


def _full_spec(shape):
    return pl.BlockSpec(shape, lambda i: tuple(0 for _ in shape))


def kernel(h, edge_index, W1, b1, W2, b2, W3, b3):
    src = edge_index[0]
    dst = edge_index[1]
    pad = E_PAD - N_EDGES
    pad_idx = (N_NODES + jnp.arange(pad, dtype=jnp.int32)
               % (N_PAD - N_NODES)).astype(src.dtype)
    srcf = jnp.concatenate([src, pad_idx]).astype(jnp.int32)
    srcc = srcf.reshape(C_TOTAL, CHUNK)
    dstf = jnp.concatenate([dst, pad_idx]).astype(jnp.int32)
    dstc = dstf.reshape(C_TOTAL, CHUNK)

    h8 = jnp.pad(h, ((0, N_PAD - N_NODES), (0, 2)))
    W1p = jnp.pad(W1, ((0, 2), (0, 0)))
    zeros16 = jnp.zeros((N_PAD, F), jnp.float32)
    zeros8 = jnp.zeros((N_PAD, 8), jnp.float32)
    zerosd = jnp.zeros((N_PAD, DW), jnp.float32)
    onesd = (jnp.zeros((2, CHUNK, DW), jnp.float32)
             .at[0, :, 0].set(1.0).at[1, :, 4].set(1.0))

    deg = _degrees(srcc, dstc, onesd, zerosd)

    y1, ns, nd = pl.pallas_call(
        _tc1_body,
        grid=(GRID,),
        in_specs=[_part_spec(DW), _row_spec(8)],
        out_specs=[_row_spec(8), _row_spec(1), _row_spec(1)],
        out_shape=[jax.ShapeDtypeStruct((N_PAD, 8), jnp.float32),
                   jax.ShapeDtypeStruct((N_PAD, 1), jnp.float32),
                   jax.ShapeDtypeStruct((N_PAD, 1), jnp.float32)],
    )(deg, h8)

    agg1 = _edge_pass(8, GD, y1, srcf, dstf, zeros8)

    y2 = pl.pallas_call(
        _tc2_body,
        grid=(GRID,),
        in_specs=[_part_spec(8), _row_spec(1), _full_spec((1, F)),
                  _row_spec(1), _full_spec((8, F)), _full_spec((F, F))],
        out_specs=_row_spec(F),
        out_shape=jax.ShapeDtypeStruct((N_PAD, F), jnp.float32),
    )(agg1, nd, b1.reshape(1, F), ns, W1p, W2)

    agg2 = _edge_pass(F, G, y2, srcf, dstf, zeros16)

    o = pl.pallas_call(
        _tc3_body,
        grid=(GRID,),
        in_specs=[_part_spec(F), _row_spec(1), _full_spec((1, F)),
                  _full_spec((1, F)), _full_spec((1, 1))],
        out_specs=_row_spec(1),
        out_shape=jax.ShapeDtypeStruct((N_PAD, 1), jnp.float32),
    )(agg2, nd, b2.reshape(1, F), W3.reshape(1, F), b3.reshape(1, 1))

    return o[:N_NODES, 0]


# speedup vs baseline: 38.3524x; 1.0367x over previous
import jax
import jax.numpy as jnp
from jax import lax
from jax.experimental import pallas as pl
from jax.experimental.pallas import tpu as pltpu
from jax.experimental.pallas import tpu_sc as plsc

N_NODES = 100000
N_EDGES = 6400000

NC = 2
NS = 16
NW = NC * NS
CHUNK = 128
G = 4
GD = 8
N_PAD = 100352
DUMMY = N_NODES

E_PAD = 6422528
C_TOTAL = E_PAD // CHUNK
C_TILE = C_TOTAL // NW
ITERS = C_TILE // G
ITERS_D = C_TILE // GD

F = 16
BLK = 2048
GRID = N_PAD // BLK

_sc_params = pltpu.CompilerParams(use_tc_tiling_on_sc=False)


def _sc_mesh():
    return plsc.VectorSubcoreMesh(core_axis_name="c", subcore_axis_name="s",
                                  num_cores=NC, num_subcores=NS)


def _make_edge_pass_body(width, g, iters):
    gc = g * CHUNK

    def _edge_pass_body(table, srcf, dstf, zeros, out,
                        src_v, dst_v, rows, acc, gsem0, gsem1, ssem0, ssem1):
        c = lax.axis_index("c")
        s = lax.axis_index("s")
        wid = s * NC + c
        rps = N_PAD // NS
        pltpu.sync_copy(zeros.at[pl.ds(s * rps, rps)],
                        acc.at[pl.ds(s * rps, rps)])
        plsc.subcore_barrier()
        base = wid * C_TILE
        gsems = (gsem0, gsem1)
        ssems = (ssem0, ssem1)

        def fire_batch(b, slot):
            pltpu.sync_copy(srcf.at[pl.ds((base + b * g) * CHUNK, gc)],
                            src_v.at[slot])
            pltpu.sync_copy(dstf.at[pl.ds((base + b * g) * CHUNK, gc)],
                            dst_v.at[slot])
            pltpu.async_copy(table.at[src_v.at[slot]], rows.at[slot],
                             gsems[slot])

        def wait_gathers(slot):
            pltpu.make_async_copy(table.at[src_v.at[slot]],
                                  rows.at[slot], gsems[slot]).wait()

        def fire_scatters(slot):
            pltpu.async_copy(rows.at[slot], acc.at[dst_v.at[slot]],
                             ssems[slot], add=True)

        def wait_scatters(slot):
            pltpu.make_async_copy(rows.at[slot], acc.at[dst_v.at[slot]],
                                  ssems[slot]).wait()

        def body(it2, carry):
            b0 = 2 * it2

            @pl.when(it2 > 0)
            def _():
                wait_scatters(0)
            fire_batch(b0, 0)

            @pl.when(it2 > 0)
            def _():
                wait_scatters(1)
            fire_batch(b0 + 1, 1)

            wait_gathers(0)
            fire_scatters(0)
            wait_gathers(1)
            fire_scatters(1)
            return carry

        lax.fori_loop(0, iters // 2, body, 0)
        wait_scatters(0)
        wait_scatters(1)
        plsc.subcore_barrier()
        pltpu.sync_copy(acc.at[pl.ds(s * rps, rps)],
                        out.at[c, pl.ds(s * rps, rps)])

    return _edge_pass_body


_lazy_cache = {}


def _edge_pass(width, g, *args):
    key = ("edge", width)
    if key not in _lazy_cache:
        _lazy_cache[key] = pl.kernel(
            _make_edge_pass_body(width, g, C_TILE // g),
            out_type=jax.ShapeDtypeStruct((NC, N_PAD, width), jnp.float32),
            mesh=_sc_mesh(),
            compiler_params=_sc_params,
            scratch_types=[
                pltpu.VMEM((2, g * CHUNK), jnp.int32),
                pltpu.VMEM((2, g * CHUNK), jnp.int32),
                pltpu.VMEM((2, g * CHUNK, width), jnp.float32),
                pltpu.VMEM_SHARED((N_PAD, width), jnp.float32),
                pltpu.SemaphoreType.DMA,
                pltpu.SemaphoreType.DMA,
                pltpu.SemaphoreType.DMA,
                pltpu.SemaphoreType.DMA,
            ],
        )
    return _lazy_cache[key](*args)


DW = 8


def _degrees_body(srcc, dstc, ones_hbm, zerosd, deg_out,
                  src_v, dst_v, ones_v, deg, sem0, sem1):
    c = lax.axis_index("c")
    s = lax.axis_index("s")
    wid = s * NC + c
    rps = N_PAD // NS
    pltpu.sync_copy(zerosd.at[pl.ds(s * rps, rps)], deg.at[pl.ds(s * rps, rps)])
    pltpu.sync_copy(ones_hbm, ones_v)
    plsc.subcore_barrier()
    base = wid * C_TILE
    sems = (sem0, sem1)

    def fire_batch(b, slot):
        off = base + b * GD
        pltpu.sync_copy(srcc.at[pl.ds(off, GD)], src_v.at[slot])
        pltpu.sync_copy(dstc.at[pl.ds(off, GD)], dst_v.at[slot])
        for j in range(GD):
            pltpu.async_copy(ones_v.at[0], deg.at[src_v.at[slot, j]],
                             sems[slot], add=True)
            pltpu.async_copy(ones_v.at[1], deg.at[dst_v.at[slot, j]],
                             sems[slot], add=True)

    def wait_batch(slot):
        for j in range(GD):
            pltpu.make_async_copy(ones_v.at[0], deg.at[src_v.at[slot, j]],
                                  sems[slot]).wait()
            pltpu.make_async_copy(ones_v.at[1], deg.at[dst_v.at[slot, j]],
                                  sems[slot]).wait()

    def body(it2, carry):
        b0 = 2 * it2

        @pl.when(it2 > 0)
        def _():
            wait_batch(0)
        fire_batch(b0, 0)

        @pl.when(it2 > 0)
        def _():
            wait_batch(1)
        fire_batch(b0 + 1, 1)
        return carry

    lax.fori_loop(0, ITERS_D // 2, body, 0)
    wait_batch(0)
    wait_batch(1)
    plsc.subcore_barrier()
    pltpu.sync_copy(deg.at[pl.ds(s * rps, rps)], deg_out.at[c, pl.ds(s * rps, rps)])


def _degrees(*args):
    if "deg" not in _lazy_cache:
        _lazy_cache["deg"] = pl.kernel(
            _degrees_body,
            out_type=jax.ShapeDtypeStruct((NC, N_PAD, DW), jnp.float32),
            mesh=_sc_mesh(),
            compiler_params=_sc_params,
            scratch_types=[
                pltpu.VMEM((2, GD, CHUNK), jnp.int32),
                pltpu.VMEM((2, GD, CHUNK), jnp.int32),
                pltpu.VMEM((2, CHUNK, DW), jnp.float32),
                pltpu.VMEM_SHARED((N_PAD, DW), jnp.float32),
                pltpu.SemaphoreType.DMA,
                pltpu.SemaphoreType.DMA,
            ],
        )
    return _lazy_cache["deg"](*args)


def _tc1_body(deg_ref, h_ref, y1_ref, ns_ref, nd_ref):
    deg_o = deg_ref[0, :, 0:1] + deg_ref[1, :, 0:1]
    deg_i = deg_ref[0, :, 4:5] + deg_ref[1, :, 4:5]
    ns = jnp.where(deg_o > 0, lax.rsqrt(jnp.maximum(deg_o, 1.0)), 0.0)
    nd = jnp.where(deg_i > 0, lax.rsqrt(jnp.maximum(deg_i, 1.0)), 0.0)
    y1_ref[...] = h_ref[...] * ns
    ns_ref[...] = ns
    nd_ref[...] = nd


def _tc2_body(agg_ref, nd_ref, b1_ref, ns_ref, w1_ref, y2_ref):
    x = jnp.dot((agg_ref[0] + agg_ref[1]) * nd_ref[...], w1_ref[...],
                preferred_element_type=jnp.float32) + b1_ref[...]
    x = jnp.maximum(x, 0.0)
    y2_ref[...] = x * ns_ref[...]


def _tc3_body(agg_ref, nd_ref, b2_ref, w2_ref, w3_ref, b3_ref, o_ref):
    x = jnp.dot((agg_ref[0] + agg_ref[1]) * nd_ref[...], w2_ref[...],
                preferred_element_type=jnp.float32) + b2_ref[...]
    x = jnp.maximum(x, 0.0)
    o_ref[...] = jnp.dot(x, w3_ref[...],
                         preferred_element_type=jnp.float32) + b3_ref[...]


def _part_spec(width):
    return pl.BlockSpec((NC, BLK, width), lambda i: (0, i, 0))


def _row_spec(width):
    return pl.BlockSpec((BLK, width), lambda i: (i, 0))


def _full_spec(shape):
    return pl.BlockSpec(shape, lambda i: tuple(0 for _ in shape))


def kernel(h, edge_index, W1, b1, W2, b2, W3, b3):
    src = edge_index[0]
    dst = edge_index[1]
    pad = E_PAD - N_EDGES
    pad_idx = (N_NODES + jnp.arange(pad, dtype=jnp.int32)
               % (N_PAD - N_NODES)).astype(src.dtype)
    srcf = jnp.concatenate([src, pad_idx]).astype(jnp.int32)
    srcc = srcf.reshape(C_TOTAL, CHUNK)
    dstf = jnp.concatenate([dst, pad_idx]).astype(jnp.int32)
    dstc = dstf.reshape(C_TOTAL, CHUNK)

    h8 = jnp.pad(h, ((0, N_PAD - N_NODES), (0, 2)))
    W1p = jnp.pad(W1, ((0, 2), (0, 0)))
    zeros16 = jnp.zeros((N_PAD, F), jnp.float32)
    zeros8 = jnp.zeros((N_PAD, 8), jnp.float32)
    zerosd = jnp.zeros((N_PAD, DW), jnp.float32)
    onesd = (jnp.zeros((2, CHUNK, DW), jnp.float32)
             .at[0, :, 0].set(1.0).at[1, :, 4].set(1.0))

    deg = _degrees(srcc, dstc, onesd, zerosd)

    y1, ns, nd = pl.pallas_call(
        _tc1_body,
        grid=(GRID,),
        in_specs=[_part_spec(DW), _row_spec(8)],
        out_specs=[_row_spec(8), _row_spec(1), _row_spec(1)],
        out_shape=[jax.ShapeDtypeStruct((N_PAD, 8), jnp.float32),
                   jax.ShapeDtypeStruct((N_PAD, 1), jnp.float32),
                   jax.ShapeDtypeStruct((N_PAD, 1), jnp.float32)],
    )(deg, h8)

    agg1 = _edge_pass(8, GD, y1, srcf, dstf, zeros8)

    y2 = pl.pallas_call(
        _tc2_body,
        grid=(GRID,),
        in_specs=[_part_spec(8), _row_spec(1), _full_spec((1, F)),
                  _row_spec(1), _full_spec((8, F))],
        out_specs=_row_spec(F),
        out_shape=jax.ShapeDtypeStruct((N_PAD, F), jnp.float32),
    )(agg1, nd, b1.reshape(1, F), ns, W1p)

    agg2 = _edge_pass(F, G, y2, srcf, dstf, zeros16)

    o = pl.pallas_call(
        _tc3_body,
        grid=(GRID,),
        in_specs=[_part_spec(F), _row_spec(1), _full_spec((1, F)),
                  _full_spec((F, F)), _full_spec((F, 1)), _full_spec((1, 1))],
        out_specs=_row_spec(1),
        out_shape=jax.ShapeDtypeStruct((N_PAD, 1), jnp.float32),
    )(agg2, nd, b2.reshape(1, F), W2, W3, b3.reshape(1, 1))

    return o[:N_NODES, 0]


# speedup vs baseline: 46.7638x; 1.2193x over previous
import jax
import jax.numpy as jnp
from jax import lax
from jax.experimental import pallas as pl
from jax.experimental.pallas import tpu as pltpu
from jax.experimental.pallas import tpu_sc as plsc

N_NODES = 100000
N_EDGES = 6400000

NC = 2
NS = 16
NW = NC * NS
CHUNK = 128
G = 4
GD = 8
N_PAD = 100352
DUMMY = N_NODES

E_PAD = 6488064
C_TOTAL = E_PAD // CHUNK
C_TILE = C_TOTAL // NW
ITERS = C_TILE // G
ITERS_D = C_TILE // GD

F = 16
BLK = 2048
GRID = N_PAD // BLK

_sc_params = pltpu.CompilerParams(use_tc_tiling_on_sc=False)


def _sc_mesh():
    return plsc.VectorSubcoreMesh(core_axis_name="c", subcore_axis_name="s",
                                  num_cores=NC, num_subcores=NS)


def _make_edge_pass_body(width, g, iters):
    gc = g * CHUNK

    def _edge_pass_body(table, srcf, dstf, zeros, out,
                        src_v, dst_v, rows, acc,
                        gsem0, gsem1, ssem0, ssem1, is0, is1, is2, is3):
        c = lax.axis_index("c")
        s = lax.axis_index("s")
        wid = s * NC + c
        rps = N_PAD // NS
        pltpu.sync_copy(zeros.at[pl.ds(s * rps, rps)],
                        acc.at[pl.ds(s * rps, rps)])
        plsc.subcore_barrier()
        base = wid * C_TILE
        gsems = (gsem0, gsem1)
        ssems = (ssem0, ssem1)
        isems = (is0, is1, is2, is3)

        def fire_idx(b, islot):
            off = (base + b * g) * CHUNK
            pltpu.async_copy(srcf.at[pl.ds(off, gc)], src_v.at[islot],
                             isems[islot])
            pltpu.async_copy(dstf.at[pl.ds(off, gc)], dst_v.at[islot],
                             isems[islot])

        def wait_idx(b, islot):
            off = (base + b * g) * CHUNK
            pltpu.make_async_copy(srcf.at[pl.ds(off, gc)], src_v.at[islot],
                                  isems[islot]).wait()
            pltpu.make_async_copy(dstf.at[pl.ds(off, gc)], dst_v.at[islot],
                                  isems[islot]).wait()

        def fire_gather(islot, rslot):
            pltpu.async_copy(table.at[src_v.at[islot]], rows.at[rslot],
                             gsems[rslot])

        def wait_gather(islot, rslot):
            pltpu.make_async_copy(table.at[src_v.at[islot]], rows.at[rslot],
                                  gsems[rslot]).wait()

        def fire_scatter(islot, rslot):
            pltpu.async_copy(rows.at[rslot], acc.at[dst_v.at[islot]],
                             ssems[rslot], add=True)

        def wait_scatter(islot, rslot):
            pltpu.make_async_copy(rows.at[rslot], acc.at[dst_v.at[islot]],
                                  ssems[rslot]).wait()

        fire_idx(0, 0)
        fire_idx(1, 1)
        last = iters // 4 - 1

        def body(it4, carry):
            b0 = 4 * it4

            @pl.when(it4 > 0)
            def _():
                wait_scatter(2, 0)
            fire_idx(b0 + 2, 2)
            wait_idx(b0, 0)
            fire_gather(0, 0)

            @pl.when(it4 > 0)
            def _():
                wait_scatter(3, 1)
            fire_idx(b0 + 3, 3)
            wait_idx(b0 + 1, 1)
            fire_gather(1, 1)

            wait_gather(0, 0)
            fire_scatter(0, 0)
            wait_gather(1, 1)
            fire_scatter(1, 1)

            wait_scatter(0, 0)

            @pl.when(it4 < last)
            def _():
                fire_idx(b0 + 4, 0)
            wait_idx(b0 + 2, 2)
            fire_gather(2, 0)

            wait_scatter(1, 1)

            @pl.when(it4 < last)
            def _():
                fire_idx(b0 + 5, 1)
            wait_idx(b0 + 3, 3)
            fire_gather(3, 1)

            wait_gather(2, 0)
            fire_scatter(2, 0)
            wait_gather(3, 1)
            fire_scatter(3, 1)
            return carry

        lax.fori_loop(0, iters // 4, body, 0)
        wait_scatter(2, 0)
        wait_scatter(3, 1)
        plsc.subcore_barrier()
        pltpu.sync_copy(acc.at[pl.ds(s * rps, rps)],
                        out.at[c, pl.ds(s * rps, rps)])

    return _edge_pass_body


_lazy_cache = {}


def _edge_pass(width, g, *args):
    key = ("edge", width)
    if key not in _lazy_cache:
        _lazy_cache[key] = pl.kernel(
            _make_edge_pass_body(width, g, C_TILE // g),
            out_type=jax.ShapeDtypeStruct((NC, N_PAD, width), jnp.float32),
            mesh=_sc_mesh(),
            compiler_params=_sc_params,
            scratch_types=[
                pltpu.VMEM((4, g * CHUNK), jnp.int32),
                pltpu.VMEM((4, g * CHUNK), jnp.int32),
                pltpu.VMEM((2, g * CHUNK, width), jnp.float32),
                pltpu.VMEM_SHARED((N_PAD, width), jnp.float32),
                pltpu.SemaphoreType.DMA,
                pltpu.SemaphoreType.DMA,
                pltpu.SemaphoreType.DMA,
                pltpu.SemaphoreType.DMA,
                pltpu.SemaphoreType.DMA,
                pltpu.SemaphoreType.DMA,
                pltpu.SemaphoreType.DMA,
                pltpu.SemaphoreType.DMA,
            ],
        )
    return _lazy_cache[key](*args)


DW = 8


def _degrees_body(srcc, dstc, ones_hbm, zerosd, deg_out,
                  src_v, dst_v, ones_v, deg, sem0, sem1):
    c = lax.axis_index("c")
    s = lax.axis_index("s")
    wid = s * NC + c
    rps = N_PAD // NS
    pltpu.sync_copy(zerosd.at[pl.ds(s * rps, rps)], deg.at[pl.ds(s * rps, rps)])
    pltpu.sync_copy(ones_hbm, ones_v)
    plsc.subcore_barrier()
    base = wid * C_TILE
    sems = (sem0, sem1)

    def fire_batch(b, slot):
        off = base + b * GD
        pltpu.sync_copy(srcc.at[pl.ds(off, GD)], src_v.at[slot])
        pltpu.sync_copy(dstc.at[pl.ds(off, GD)], dst_v.at[slot])
        for j in range(GD):
            pltpu.async_copy(ones_v.at[0], deg.at[src_v.at[slot, j]],
                             sems[slot], add=True)
            pltpu.async_copy(ones_v.at[1], deg.at[dst_v.at[slot, j]],
                             sems[slot], add=True)

    def wait_batch(slot):
        for j in range(GD):
            pltpu.make_async_copy(ones_v.at[0], deg.at[src_v.at[slot, j]],
                                  sems[slot]).wait()
            pltpu.make_async_copy(ones_v.at[1], deg.at[dst_v.at[slot, j]],
                                  sems[slot]).wait()

    def body(it2, carry):
        b0 = 2 * it2

        @pl.when(it2 > 0)
        def _():
            wait_batch(0)
        fire_batch(b0, 0)

        @pl.when(it2 > 0)
        def _():
            wait_batch(1)
        fire_batch(b0 + 1, 1)
        return carry

    lax.fori_loop(0, ITERS_D // 2, body, 0)
    wait_batch(0)
    wait_batch(1)
    plsc.subcore_barrier()
    pltpu.sync_copy(deg.at[pl.ds(s * rps, rps)], deg_out.at[c, pl.ds(s * rps, rps)])


def _degrees(*args):
    if "deg" not in _lazy_cache:
        _lazy_cache["deg"] = pl.kernel(
            _degrees_body,
            out_type=jax.ShapeDtypeStruct((NC, N_PAD, DW), jnp.float32),
            mesh=_sc_mesh(),
            compiler_params=_sc_params,
            scratch_types=[
                pltpu.VMEM((2, GD, CHUNK), jnp.int32),
                pltpu.VMEM((2, GD, CHUNK), jnp.int32),
                pltpu.VMEM((2, CHUNK, DW), jnp.float32),
                pltpu.VMEM_SHARED((N_PAD, DW), jnp.float32),
                pltpu.SemaphoreType.DMA,
                pltpu.SemaphoreType.DMA,
            ],
        )
    return _lazy_cache["deg"](*args)


def _tc1_body(deg_ref, h_ref, y1_ref, ns_ref, nd_ref):
    deg_o = deg_ref[0, :, 0:1] + deg_ref[1, :, 0:1]
    deg_i = deg_ref[0, :, 4:5] + deg_ref[1, :, 4:5]
    ns = jnp.where(deg_o > 0, lax.rsqrt(jnp.maximum(deg_o, 1.0)), 0.0)
    nd = jnp.where(deg_i > 0, lax.rsqrt(jnp.maximum(deg_i, 1.0)), 0.0)
    y1_ref[...] = h_ref[...] * ns
    ns_ref[...] = ns
    nd_ref[...] = nd


def _tc2_body(agg_ref, nd_ref, b1_ref, ns_ref, w1_ref, y2_ref):
    x = jnp.dot((agg_ref[0] + agg_ref[1]) * nd_ref[...], w1_ref[...],
                preferred_element_type=jnp.float32) + b1_ref[...]
    x = jnp.maximum(x, 0.0)
    y2_ref[...] = x * ns_ref[...]


def _tc3_body(agg_ref, nd_ref, b2_ref, w2_ref, w3_ref, b3_ref, o_ref):
    x = jnp.dot((agg_ref[0] + agg_ref[1]) * nd_ref[...], w2_ref[...],
                preferred_element_type=jnp.float32) + b2_ref[...]
    x = jnp.maximum(x, 0.0)
    o_ref[...] = jnp.dot(x, w3_ref[...],
                         preferred_element_type=jnp.float32) + b3_ref[...]


def _part_spec(width):
    return pl.BlockSpec((NC, BLK, width), lambda i: (0, i, 0))


def _row_spec(width):
    return pl.BlockSpec((BLK, width), lambda i: (i, 0))


def _full_spec(shape):
    return pl.BlockSpec(shape, lambda i: tuple(0 for _ in shape))


def kernel(h, edge_index, W1, b1, W2, b2, W3, b3):
    src = edge_index[0]
    dst = edge_index[1]
    pad = E_PAD - N_EDGES
    pad_idx = (N_NODES + jnp.arange(pad, dtype=jnp.int32)
               % (N_PAD - N_NODES)).astype(src.dtype)
    srcf = jnp.concatenate([src, pad_idx]).astype(jnp.int32)
    srcc = srcf.reshape(C_TOTAL, CHUNK)
    dstf = jnp.concatenate([dst, pad_idx]).astype(jnp.int32)
    dstc = dstf.reshape(C_TOTAL, CHUNK)

    h8 = jnp.pad(h, ((0, N_PAD - N_NODES), (0, 2)))
    W1p = jnp.pad(W1, ((0, 2), (0, 0)))
    zeros16 = jnp.zeros((N_PAD, F), jnp.float32)
    zeros8 = jnp.zeros((N_PAD, 8), jnp.float32)
    zerosd = jnp.zeros((N_PAD, DW), jnp.float32)
    onesd = (jnp.zeros((2, CHUNK, DW), jnp.float32)
             .at[0, :, 0].set(1.0).at[1, :, 4].set(1.0))

    deg = _degrees(srcc, dstc, onesd, zerosd)

    y1, ns, nd = pl.pallas_call(
        _tc1_body,
        grid=(GRID,),
        in_specs=[_part_spec(DW), _row_spec(8)],
        out_specs=[_row_spec(8), _row_spec(1), _row_spec(1)],
        out_shape=[jax.ShapeDtypeStruct((N_PAD, 8), jnp.float32),
                   jax.ShapeDtypeStruct((N_PAD, 1), jnp.float32),
                   jax.ShapeDtypeStruct((N_PAD, 1), jnp.float32)],
    )(deg, h8)

    agg1 = _edge_pass(8, G, y1, srcf, dstf, zeros8)

    y2 = pl.pallas_call(
        _tc2_body,
        grid=(GRID,),
        in_specs=[_part_spec(8), _row_spec(1), _full_spec((1, F)),
                  _row_spec(1), _full_spec((8, F))],
        out_specs=_row_spec(F),
        out_shape=jax.ShapeDtypeStruct((N_PAD, F), jnp.float32),
    )(agg1, nd, b1.reshape(1, F), ns, W1p)

    agg2 = _edge_pass(F, G, y2, srcf, dstf, zeros16)

    o = pl.pallas_call(
        _tc3_body,
        grid=(GRID,),
        in_specs=[_part_spec(F), _row_spec(1), _full_spec((1, F)),
                  _full_spec((F, F)), _full_spec((F, 1)), _full_spec((1, 1))],
        out_specs=_row_spec(1),
        out_shape=jax.ShapeDtypeStruct((N_PAD, 1), jnp.float32),
    )(agg2, nd, b2.reshape(1, F), W2, W3, b3.reshape(1, 1))

    return o[:N_NODES, 0]


# speedup vs baseline: 46.9786x; 1.0046x over previous
import jax
import jax.numpy as jnp
from jax import lax
from jax.experimental import pallas as pl
from jax.experimental.pallas import tpu as pltpu
from jax.experimental.pallas import tpu_sc as plsc

N_NODES = 100000
N_EDGES = 6400000

NC = 2
NS = 16
NW = NC * NS
CHUNK = 128
G = 4
GD = 8
N_PAD = 100352
DUMMY = N_NODES

E_PAD = 6488064
C_TOTAL = E_PAD // CHUNK
C_TILE = C_TOTAL // NW
ITERS = C_TILE // G
ITERS_D = C_TILE // GD

F = 16
BLK = 2048
GRID = N_PAD // BLK

_sc_params = pltpu.CompilerParams(use_tc_tiling_on_sc=False)


def _sc_mesh():
    return plsc.VectorSubcoreMesh(core_axis_name="c", subcore_axis_name="s",
                                  num_cores=NC, num_subcores=NS)


def _make_edge_pass_body(width, g, iters):
    gc = g * CHUNK

    def _edge_pass_body(table, srcf, dstf, zeros, out,
                        src_v, dst_v, rows, acc,
                        gsem0, gsem1, ssem0, ssem1, is0, is1, is2, is3):
        c = lax.axis_index("c")
        s = lax.axis_index("s")
        wid = s * NC + c
        rps = N_PAD // NS
        pltpu.sync_copy(zeros.at[pl.ds(s * rps, rps)],
                        acc.at[pl.ds(s * rps, rps)])
        plsc.subcore_barrier()
        base = wid * C_TILE
        gsems = (gsem0, gsem1)
        ssems = (ssem0, ssem1)
        isems = (is0, is1, is2, is3)

        def fire_idx(b, islot):
            off = (base + b * g) * CHUNK
            pltpu.async_copy(srcf.at[pl.ds(off, gc)], src_v.at[islot],
                             isems[islot])
            pltpu.async_copy(dstf.at[pl.ds(off, gc)], dst_v.at[islot],
                             isems[islot])

        def wait_idx(b, islot):
            off = (base + b * g) * CHUNK
            pltpu.make_async_copy(srcf.at[pl.ds(off, gc)], src_v.at[islot],
                                  isems[islot]).wait()
            pltpu.make_async_copy(dstf.at[pl.ds(off, gc)], dst_v.at[islot],
                                  isems[islot]).wait()

        def fire_gather(islot, rslot):
            pltpu.async_copy(table.at[src_v.at[islot]], rows.at[rslot],
                             gsems[rslot])

        def wait_gather(islot, rslot):
            pltpu.make_async_copy(table.at[src_v.at[islot]], rows.at[rslot],
                                  gsems[rslot]).wait()

        def fire_scatter(islot, rslot):
            pltpu.async_copy(rows.at[rslot], acc.at[dst_v.at[islot]],
                             ssems[rslot], add=True)

        def wait_scatter(islot, rslot):
            pltpu.make_async_copy(rows.at[rslot], acc.at[dst_v.at[islot]],
                                  ssems[rslot]).wait()

        fire_idx(0, 0)
        fire_idx(1, 1)
        last = iters // 4 - 1

        def body(it4, carry):
            b0 = 4 * it4

            @pl.when(it4 > 0)
            def _():
                wait_scatter(2, 0)
            fire_idx(b0 + 2, 2)
            wait_idx(b0, 0)
            fire_gather(0, 0)

            @pl.when(it4 > 0)
            def _():
                wait_scatter(3, 1)
            fire_idx(b0 + 3, 3)
            wait_idx(b0 + 1, 1)
            fire_gather(1, 1)

            wait_gather(0, 0)
            fire_scatter(0, 0)
            wait_gather(1, 1)
            fire_scatter(1, 1)

            wait_scatter(0, 0)

            @pl.when(it4 < last)
            def _():
                fire_idx(b0 + 4, 0)
            wait_idx(b0 + 2, 2)
            fire_gather(2, 0)

            wait_scatter(1, 1)

            @pl.when(it4 < last)
            def _():
                fire_idx(b0 + 5, 1)
            wait_idx(b0 + 3, 3)
            fire_gather(3, 1)

            wait_gather(2, 0)
            fire_scatter(2, 0)
            wait_gather(3, 1)
            fire_scatter(3, 1)
            return carry

        lax.fori_loop(0, iters // 4, body, 0)
        wait_scatter(2, 0)
        wait_scatter(3, 1)
        plsc.subcore_barrier()
        pltpu.sync_copy(acc.at[pl.ds(s * rps, rps)],
                        out.at[c, pl.ds(s * rps, rps)])

    return _edge_pass_body


_lazy_cache = {}


def _edge_pass(width, g, *args):
    key = ("edge", width)
    if key not in _lazy_cache:
        _lazy_cache[key] = pl.kernel(
            _make_edge_pass_body(width, g, C_TILE // g),
            out_type=jax.ShapeDtypeStruct((NC, N_PAD, width), jnp.float32),
            mesh=_sc_mesh(),
            compiler_params=_sc_params,
            scratch_types=[
                pltpu.VMEM((4, g * CHUNK), jnp.int32),
                pltpu.VMEM((4, g * CHUNK), jnp.int32),
                pltpu.VMEM((2, g * CHUNK, width), jnp.float32),
                pltpu.VMEM_SHARED((N_PAD, width), jnp.float32),
                pltpu.SemaphoreType.DMA,
                pltpu.SemaphoreType.DMA,
                pltpu.SemaphoreType.DMA,
                pltpu.SemaphoreType.DMA,
                pltpu.SemaphoreType.DMA,
                pltpu.SemaphoreType.DMA,
                pltpu.SemaphoreType.DMA,
                pltpu.SemaphoreType.DMA,
            ],
        )
    return _lazy_cache[key](*args)


DW = 8


def _degrees_body(srcf, dstf, ones_hbm, zerosd, deg_out,
                  src_v, dst_v, ones_v, deg, sem0, sem1):
    c = lax.axis_index("c")
    s = lax.axis_index("s")
    wid = s * NC + c
    rps = N_PAD // NS
    pltpu.sync_copy(zerosd.at[pl.ds(s * rps, rps)], deg.at[pl.ds(s * rps, rps)])
    pltpu.sync_copy(ones_hbm, ones_v)
    plsc.subcore_barrier()
    base = wid * C_TILE
    sems = (sem0, sem1)
    gcd_ = GD * CHUNK

    def fire_batch(b, slot):
        off = (base + b * GD) * CHUNK
        pltpu.sync_copy(srcf.at[pl.ds(off, gcd_)], src_v.at[slot])
        pltpu.sync_copy(dstf.at[pl.ds(off, gcd_)], dst_v.at[slot])
        pltpu.async_copy(ones_v.at[0], deg.at[src_v.at[slot]],
                         sems[slot], add=True)
        pltpu.async_copy(ones_v.at[1], deg.at[dst_v.at[slot]],
                         sems[slot], add=True)

    def wait_batch(slot):
        pltpu.make_async_copy(ones_v.at[0], deg.at[src_v.at[slot]],
                              sems[slot]).wait()
        pltpu.make_async_copy(ones_v.at[1], deg.at[dst_v.at[slot]],
                              sems[slot]).wait()

    def body(it2, carry):
        b0 = 2 * it2

        @pl.when(it2 > 0)
        def _():
            wait_batch(0)
        fire_batch(b0, 0)

        @pl.when(it2 > 0)
        def _():
            wait_batch(1)
        fire_batch(b0 + 1, 1)
        return carry

    lax.fori_loop(0, ITERS_D // 2, body, 0)
    wait_batch(0)
    wait_batch(1)
    plsc.subcore_barrier()
    pltpu.sync_copy(deg.at[pl.ds(s * rps, rps)], deg_out.at[c, pl.ds(s * rps, rps)])


def _degrees(*args):
    if "deg" not in _lazy_cache:
        _lazy_cache["deg"] = pl.kernel(
            _degrees_body,
            out_type=jax.ShapeDtypeStruct((NC, N_PAD, DW), jnp.float32),
            mesh=_sc_mesh(),
            compiler_params=_sc_params,
            scratch_types=[
                pltpu.VMEM((2, GD * CHUNK), jnp.int32),
                pltpu.VMEM((2, GD * CHUNK), jnp.int32),
                pltpu.VMEM((2, GD * CHUNK, DW), jnp.float32),
                pltpu.VMEM_SHARED((N_PAD, DW), jnp.float32),
                pltpu.SemaphoreType.DMA,
                pltpu.SemaphoreType.DMA,
            ],
        )
    return _lazy_cache["deg"](*args)


def _tc1_body(deg_ref, h_ref, y1_ref, ns_ref, nd_ref):
    deg_o = deg_ref[0, :, 0:1] + deg_ref[1, :, 0:1]
    deg_i = deg_ref[0, :, 4:5] + deg_ref[1, :, 4:5]
    ns = jnp.where(deg_o > 0, lax.rsqrt(jnp.maximum(deg_o, 1.0)), 0.0)
    nd = jnp.where(deg_i > 0, lax.rsqrt(jnp.maximum(deg_i, 1.0)), 0.0)
    y1_ref[...] = h_ref[...] * ns
    ns_ref[...] = ns
    nd_ref[...] = nd


def _tc2_body(agg_ref, nd_ref, b1_ref, ns_ref, w1_ref, y2_ref):
    x = jnp.dot((agg_ref[0] + agg_ref[1]) * nd_ref[...], w1_ref[...],
                preferred_element_type=jnp.float32) + b1_ref[...]
    x = jnp.maximum(x, 0.0)
    y2_ref[...] = x * ns_ref[...]


def _tc3_body(agg_ref, nd_ref, b2_ref, w2_ref, w3_ref, b3_ref, o_ref):
    x = jnp.dot((agg_ref[0] + agg_ref[1]) * nd_ref[...], w2_ref[...],
                preferred_element_type=jnp.float32) + b2_ref[...]
    x = jnp.maximum(x, 0.0)
    o_ref[...] = jnp.dot(x, w3_ref[...],
                         preferred_element_type=jnp.float32) + b3_ref[...]


def _part_spec(width):
    return pl.BlockSpec((NC, BLK, width), lambda i: (0, i, 0))


def _row_spec(width):
    return pl.BlockSpec((BLK, width), lambda i: (i, 0))


def _full_spec(shape):
    return pl.BlockSpec(shape, lambda i: tuple(0 for _ in shape))


def kernel(h, edge_index, W1, b1, W2, b2, W3, b3):
    src = edge_index[0]
    dst = edge_index[1]
    pad = E_PAD - N_EDGES
    pad_idx = (N_NODES + jnp.arange(pad, dtype=jnp.int32)
               % (N_PAD - N_NODES)).astype(src.dtype)
    srcf = jnp.concatenate([src, pad_idx]).astype(jnp.int32)
    dstf = jnp.concatenate([dst, pad_idx]).astype(jnp.int32)

    h8 = jnp.pad(h, ((0, N_PAD - N_NODES), (0, 2)))
    W1p = jnp.pad(W1, ((0, 2), (0, 0)))
    zeros16 = jnp.zeros((N_PAD, F), jnp.float32)
    zeros8 = jnp.zeros((N_PAD, 8), jnp.float32)
    zerosd = jnp.zeros((N_PAD, DW), jnp.float32)
    onesd = (jnp.zeros((2, GD * CHUNK, DW), jnp.float32)
             .at[0, :, 0].set(1.0).at[1, :, 4].set(1.0))

    deg = _degrees(srcf, dstf, onesd, zerosd)

    y1, ns, nd = pl.pallas_call(
        _tc1_body,
        grid=(GRID,),
        in_specs=[_part_spec(DW), _row_spec(8)],
        out_specs=[_row_spec(8), _row_spec(1), _row_spec(1)],
        out_shape=[jax.ShapeDtypeStruct((N_PAD, 8), jnp.float32),
                   jax.ShapeDtypeStruct((N_PAD, 1), jnp.float32),
                   jax.ShapeDtypeStruct((N_PAD, 1), jnp.float32)],
    )(deg, h8)

    agg1 = _edge_pass(8, G, y1, srcf, dstf, zeros8)

    y2 = pl.pallas_call(
        _tc2_body,
        grid=(GRID,),
        in_specs=[_part_spec(8), _row_spec(1), _full_spec((1, F)),
                  _row_spec(1), _full_spec((8, F))],
        out_specs=_row_spec(F),
        out_shape=jax.ShapeDtypeStruct((N_PAD, F), jnp.float32),
    )(agg1, nd, b1.reshape(1, F), ns, W1p)

    agg2 = _edge_pass(F, G, y2, srcf, dstf, zeros16)

    o = pl.pallas_call(
        _tc3_body,
        grid=(GRID,),
        in_specs=[_part_spec(F), _row_spec(1), _full_spec((1, F)),
                  _full_spec((F, F)), _full_spec((F, 1)), _full_spec((1, 1))],
        out_specs=_row_spec(1),
        out_shape=jax.ShapeDtypeStruct((N_PAD, 1), jnp.float32),
    )(agg2, nd, b2.reshape(1, F), W2, W3, b3.reshape(1, 1))

    return o[:N_NODES, 0]
